# baseline (speedup 1.0000x reference)
import math

import jax
import jax.numpy as jnp
from jax.experimental import pallas as pl

_N = 10000
_DEG = 32
_D = 128
_DELTA = 3.4965
_BN = 1000

_C1 = math.log(_DEG + 1) / _DELTA
_C2 = _DELTA / math.log(_DEG + 1)


def _pna_kernel(x_ref, w_ref, b_ref, o_ref):
    v0 = x_ref[:, 0:_D]
    s = v0
    sq = v0 * v0
    mx = v0
    mn = v0
    for d in range(1, _DEG):
        v = x_ref[:, d * _D : (d + 1) * _D]
        s = s + v
        sq = sq + v * v
        mx = jnp.maximum(mx, v)
        mn = jnp.minimum(mn, v)

    mean = s * (1.0 / _DEG)
    var = sq * (1.0 / _DEG) - mean * mean
    std = jnp.sqrt(jnp.maximum(var, 0.0))

    w = w_ref[...]
    we = (
        w[0 : 4 * _D, :]
        + _C1 * w[4 * _D : 8 * _D, :]
        + _C2 * w[8 * _D : 12 * _D, :]
    )

    acc = jnp.dot(mean, we[0 * _D : 1 * _D, :])
    acc += jnp.dot(mx, we[1 * _D : 2 * _D, :])
    acc += jnp.dot(mn, we[2 * _D : 3 * _D, :])
    acc += jnp.dot(std, we[3 * _D : 4 * _D, :])
    o_ref[...] = acc + b_ref[...]


def kernel(x, W, b):
    x2 = x.reshape(_N, _DEG * _D)
    b2 = b.reshape(1, _D)
    return pl.pallas_call(
        _pna_kernel,
        grid=(_N // _BN,),
        in_specs=[
            pl.BlockSpec((_BN, _DEG * _D), lambda i: (i, 0)),
            pl.BlockSpec((12 * _D, _D), lambda i: (0, 0)),
            pl.BlockSpec((1, _D), lambda i: (0, 0)),
        ],
        out_specs=pl.BlockSpec((_BN, _D), lambda i: (i, 0)),
        out_shape=jax.ShapeDtypeStruct((_N, _D), jnp.float32),
    )(x2, W, b2)


# speedup vs baseline: 3.2858x; 3.2858x over previous
import math

import jax
import jax.numpy as jnp
from jax.experimental import pallas as pl
from jax.experimental.pallas import tpu as pltpu

_N = 10000
_DEG = 32
_D = 128
_DELTA = 3.4965
_BN = 1000
_NSTEPS = _N // _BN

_C1 = math.log(_DEG + 1) / _DELTA
_C2 = _DELTA / math.log(_DEG + 1)


def _pna_kernel(x_hbm, w_ref, b_ref, o_ref, vbuf, sems):
    i = pl.program_id(0)

    def copies(step, slot):
        return [
            pltpu.make_async_copy(
                x_hbm.at[pl.ds(step * _BN, _BN), d],
                vbuf.at[slot, d],
                sems.at[slot],
            )
            for d in range(_DEG)
        ]

    @pl.when(i == 0)
    def _():
        for c in copies(0, 0):
            c.start()

    @pl.when(i + 1 < _NSTEPS)
    def _():
        for c in copies(i + 1, (i + 1) % 2):
            c.start()

    def compute(slot):
        v0 = vbuf[slot, 0]
        s = v0
        sq = v0 * v0
        mx = v0
        mn = v0
        for d in range(1, _DEG):
            v = vbuf[slot, d]
            s = s + v
            sq = sq + v * v
            mx = jnp.maximum(mx, v)
            mn = jnp.minimum(mn, v)

        mean = s * (1.0 / _DEG)
        var = sq * (1.0 / _DEG) - mean * mean
        std = jnp.sqrt(jnp.maximum(var, 0.0))

        w = w_ref[...]
        we = (
            w[0 : 4 * _D, :]
            + _C1 * w[4 * _D : 8 * _D, :]
            + _C2 * w[8 * _D : 12 * _D, :]
        )
        acc = jnp.dot(mean, we[0 * _D : 1 * _D, :])
        acc += jnp.dot(mx, we[1 * _D : 2 * _D, :])
        acc += jnp.dot(mn, we[2 * _D : 3 * _D, :])
        acc += jnp.dot(std, we[3 * _D : 4 * _D, :])
        o_ref[...] = acc + b_ref[...]

    @pl.when(i % 2 == 0)
    def _():
        for c in copies(i, 0):
            c.wait()
        compute(0)

    @pl.when(i % 2 == 1)
    def _():
        for c in copies(i, 1):
            c.wait()
        compute(1)


def kernel(x, W, b):
    b2 = b.reshape(1, _D)
    return pl.pallas_call(
        _pna_kernel,
        grid=(_NSTEPS,),
        in_specs=[
            pl.BlockSpec(memory_space=pl.ANY),
            pl.BlockSpec((12 * _D, _D), lambda i: (0, 0)),
            pl.BlockSpec((1, _D), lambda i: (0, 0)),
        ],
        out_specs=pl.BlockSpec((_BN, _D), lambda i: (i, 0)),
        out_shape=jax.ShapeDtypeStruct((_N, _D), jnp.float32),
        scratch_shapes=[
            pltpu.VMEM((2, _DEG, _BN, _D), jnp.float32),
            pltpu.SemaphoreType.DMA((2,)),
        ],
    )(x, W, b2)
